# baseline (speedup 1.0000x reference)
import jax, jax.numpy as jnp
from jax.experimental import pallas as pl

N = 10000
E = 160000
U = 64
OUT_DIM = 1
L = 2
K = 2
B = 16
NM = K + 1


def _spmm(src, dst, wn, x):
    return jax.ops.segment_sum(wn[:, None] * x[src], dst, num_segments=N)


def _gconv(inp_b, state_b, src, dst, wn, W, b, out_size):
    x = jnp.concatenate([inp_b.reshape(B, N, -1), state_b.reshape(B, N, -1)], axis=2)
    in_dim = x.shape[2]
    x0 = jnp.transpose(x, (1, 2, 0)).reshape(N, in_dim * B)
    xs = [x0]
    x1 = _spmm(src, dst, wn, x0)
    xs.append(x1)
    for _ in range(2, K + 1):
        x2 = 2.0 * _spmm(src, dst, wn, x1) - x0
        xs.append(x2)
        x0, x1 = x1, x2
    xcat = jnp.stack(xs, axis=0).reshape(NM, N, in_dim, B)
    xcat = jnp.transpose(xcat, (3, 1, 2, 0)).reshape(B * N, in_dim * NM)
    out = xcat @ W + b
    return out.reshape(B, N * out_size)


def _cell(inp_b, hx, src, dst, wn, W_ru, b_ru, W_c, b_c):
    value = jax.nn.sigmoid(_gconv(inp_b, hx, src, dst, wn, W_ru, b_ru, 2 * U))
    value = value.reshape(-1, N, 2 * U)
    r = value[:, :, :U].reshape(-1, N * U)
    u = value[:, :, U:].reshape(-1, N * U)
    c = jnp.tanh(_gconv(inp_b, r * hx, src, dst, wn, W_c, b_c, U))
    return u * hx + (1.0 - u) * c


def kernel(inputs, hidden_state, src, dst, edge_w, W_ru_0, b_ru_0, W_c_0, b_c_0, W_ru_1, b_ru_1, W_c_1, b_c_1, W_proj, b_proj):
    deg = jax.ops.segment_sum(edge_w, src, num_segments=N)
    d_inv = jnp.where(deg > 0, 1.0 / deg, 0.0)
    wn = edge_w * d_inv[src]
    params = [(W_ru_0, b_ru_0, W_c_0, b_c_0), (W_ru_1, b_ru_1, W_c_1, b_c_1)]
    hidden_states = []
    out = inputs
    for l in range(L):
        nh = _cell(out, hidden_state[l], src, dst, wn, *params[l])
        hidden_states.append(nh)
        out = nh
    hidden = jnp.stack(hidden_states, axis=0)
    projected = out.reshape(-1, U) @ W_proj + b_proj
    output = projected.reshape(-1, N * OUT_DIM)
    return (output, hidden)


# speedup vs baseline: 1.7798x; 1.7798x over previous
import functools

import jax
import jax.numpy as jnp
from jax import lax
from jax.experimental import pallas as pl
from jax.experimental.pallas import tpu as pltpu
from jax.experimental.pallas import tpu_sc as plsc

N = 10000
E = 160000
U = 64
B = 16
NTILE = 16
PT = E // NTILE
C = 80
NCHUNK = PT // C
SW = 640
GROUPS = C // 16


def _wn_body(srcs, ews, zeros_n, wn_out, accd, srcv, ewv, wnv, dval, sem):
    c = lax.axis_index("c")
    s = lax.axis_index("s")

    @pl.when(c == 0)
    def _():
        pltpu.sync_copy(srcs.at[s], srcv)
        pltpu.sync_copy(ews.at[s], ewv)

        @pl.when(s == 0)
        def _():
            pltpu.sync_copy(zeros_n, accd)

        plsc.subcore_barrier()

        def deg_chunk(k, carry):
            pltpu.sync_copy(ewv.at[k], accd.at[srcv.at[k]], add=True)
            return carry

        lax.fori_loop(0, NCHUNK, deg_chunk, 0)
        plsc.subcore_barrier()

        def wn_chunk(k, carry):
            pltpu.async_copy(accd.at[srcv.at[k]], dval, sem).wait()
            for g in range(GROUPS):
                sl = pl.ds(g * 16, 16)
                d = dval[sl]
                pos = d > 0.0
                safe = jnp.where(pos, d, 1.0)
                wnv[k, sl] = jnp.where(pos, ewv[k, sl] / safe, 0.0)
            return carry

        lax.fori_loop(0, NCHUNK, wn_chunk, 0)
        pltpu.sync_copy(wnv, wn_out.at[s])


def _spmm_body(nt, ch, x, srcs, dsts, wnb, zeros, y,
               acc, rows, srcv, dstv, wnbv, idxb, zrows, sem):
    c = lax.axis_index("c")
    s = lax.axis_index("s")
    pltpu.sync_copy(srcs.at[s], srcv)
    pltpu.sync_copy(dsts.at[s], dstv)
    pltpu.sync_copy(zeros, zrows)

    rounds = max(nt // 2, 1)

    def do_round(b):
        off = s * SW
        pltpu.sync_copy(zrows, acc.at[pl.ds(off, 400)])

        @pl.when(s < NTILE - 1)
        def _():
            pltpu.sync_copy(zrows.at[pl.ds(0, SW - 400)],
                            acc.at[pl.ds(off + 400, SW - 400)])

        plsc.subcore_barrier()
        boff = b * N

        def chunk(k, carry):
            for g in range(GROUPS):
                sl = pl.ds(g * 16, 16)
                idxb[sl] = srcv[k, sl] + boff
            pltpu.sync_copy(wnb.at[s, k], wnbv)
            pltpu.async_copy(x.at[idxb], rows, sem).wait()
            for j in range(C):
                wrow = wnbv[j]
                for cc in range(ch // 16):
                    sl = pl.ds(cc * 16, 16)
                    rows[j, sl] = rows[j, sl] * wrow
            pltpu.sync_copy(rows, acc.at[dstv.at[k]], add=True)
            return carry

        lax.fori_loop(0, NCHUNK, chunk, 0)
        plsc.subcore_barrier()
        pltpu.sync_copy(acc.at[pl.ds(off, 400)], y.at[pl.ds(boff + off, 400)])

        @pl.when(s < NTILE - 1)
        def _():
            pltpu.sync_copy(acc.at[pl.ds(off + 400, SW - 400)],
                            y.at[pl.ds(boff + off + 400, SW - 400)])

    if nt == 1:
        @pl.when(c == 0)
        def _():
            do_round(0)
    else:
        for r in range(rounds):
            do_round(2 * r + c)


@functools.lru_cache(maxsize=None)
def _mesh():
    return plsc.VectorSubcoreMesh(core_axis_name="c", subcore_axis_name="s")


@functools.lru_cache(maxsize=None)
def _make_spmm(nt, ch):
    return pl.kernel(
        functools.partial(_spmm_body, nt, ch),
        out_type=jax.ShapeDtypeStruct((nt * N, ch), jnp.float32),
        mesh=_mesh(),
        scratch_types=[
            pltpu.VMEM_SHARED((N, ch), jnp.float32),
            pltpu.VMEM((C, ch), jnp.float32),
            pltpu.VMEM((NCHUNK, C), jnp.int32),
            pltpu.VMEM((NCHUNK, C), jnp.int32),
            pltpu.VMEM((C, 16), jnp.float32),
            pltpu.VMEM((C,), jnp.int32),
            pltpu.VMEM((400, ch), jnp.float32),
            pltpu.SemaphoreType.DMA,
        ],
        compiler_params=pltpu.CompilerParams(use_tc_tiling_on_sc=False),
    )


@functools.lru_cache(maxsize=None)
def _make_wn():
    return pl.kernel(
        _wn_body,
        out_type=jax.ShapeDtypeStruct((NTILE, NCHUNK, C), jnp.float32),
        mesh=_mesh(),
        scratch_types=[
            pltpu.VMEM_SHARED((N,), jnp.float32),
            pltpu.VMEM((NCHUNK, C), jnp.int32),
            pltpu.VMEM((NCHUNK, C), jnp.float32),
            pltpu.VMEM((NCHUNK, C), jnp.float32),
            pltpu.VMEM((C,), jnp.float32),
            pltpu.SemaphoreType.DMA,
        ],
        compiler_params=pltpu.CompilerParams(use_tc_tiling_on_sc=False),
    )


def _split_mats(W, ci, in_dim):
    Wr = W.reshape(in_dim, 3, -1)
    A0 = Wr[:, 0, :] - Wr[:, 2, :]
    A1 = Wr[:, 1, :]
    A2 = 2.0 * Wr[:, 2, :]
    return [(A[:ci], A[ci:]) for A in (A0, A1, A2)]


def _gconv_dense(fields_i, fields_h, mats, bias, ci):
    out = bias
    for m in range(3):
        Ai, Ah = mats[m]
        fi = fields_i[m]
        if ci == 1:
            out = out + fi[..., None] * Ai[0]
        else:
            out = out + jnp.einsum("bnc,co->bno", fi, Ai,
                                   preferred_element_type=jnp.float32)
        out = out + jnp.einsum("bnc,co->bno", fields_h[m], Ah,
                               preferred_element_type=jnp.float32)
    return out


def kernel(inputs, hidden_state, src, dst, edge_w,
           W_ru_0, b_ru_0, W_c_0, b_c_0, W_ru_1, b_ru_1, W_c_1, b_c_1,
           W_proj, b_proj):
    src3 = src.astype(jnp.int32).reshape(NTILE, NCHUNK, C)
    dst3 = dst.astype(jnp.int32).reshape(NTILE, NCHUNK, C)
    ew3 = edge_w.reshape(NTILE, NCHUNK, C)
    zeros_n = jnp.zeros((N,), jnp.float32)
    zeros16 = jnp.zeros((400, 16), jnp.float32)
    zeros64 = jnp.zeros((400, U), jnp.float32)

    wn3 = _make_wn()(src3, ew3, zeros_n)
    wnb3 = jnp.broadcast_to(wn3[:, :, :, None], (NTILE, NCHUNK, C, 16))

    spmm64 = _make_spmm(B, U)
    spmm16 = _make_spmm(1, 16)

    def S64(xbnc):
        flat = xbnc.reshape(B * N, U)
        return spmm64(flat, src3, dst3, wnb3, zeros64).reshape(B, N, U)

    def S16(xn16):
        return spmm16(xn16, src3, dst3, wnb3, zeros16)

    x_i = inputs.reshape(B, N)
    h0 = hidden_state[0].reshape(B, N, U)
    h1 = hidden_state[1].reshape(B, N, U)

    ti = x_i.T
    Si_t = S16(ti)
    S2i_t = S16(Si_t)
    Si = Si_t.T
    S2i = S2i_t.T
    Sh0 = S64(h0)
    S2h0 = S64(Sh0)

    mats_ru0 = _split_mats(W_ru_0, 1, 1 + U)
    val = jax.nn.sigmoid(
        _gconv_dense([x_i, Si, S2i], [h0, Sh0, S2h0], mats_ru0, b_ru_0, 1))
    r0 = val[..., :U]
    u0 = val[..., U:]
    rh0 = r0 * h0
    Srh0 = S64(rh0)
    S2rh0 = S64(Srh0)
    mats_c0 = _split_mats(W_c_0, 1, 1 + U)
    c0 = jnp.tanh(
        _gconv_dense([x_i, Si, S2i], [rh0, Srh0, S2rh0], mats_c0, b_c_0, 1))
    h0n = u0 * h0 + (1.0 - u0) * c0

    Si1 = S64(h0n)
    S2i1 = S64(Si1)
    Sh1 = S64(h1)
    S2h1 = S64(Sh1)
    mats_ru1 = _split_mats(W_ru_1, U, 2 * U)
    val1 = jax.nn.sigmoid(
        _gconv_dense([h0n, Si1, S2i1], [h1, Sh1, S2h1], mats_ru1, b_ru_1, U))
    r1 = val1[..., :U]
    u1 = val1[..., U:]
    rh1 = r1 * h1
    Srh1 = S64(rh1)
    S2rh1 = S64(Srh1)
    mats_c1 = _split_mats(W_c_1, U, 2 * U)
    c1 = jnp.tanh(
        _gconv_dense([h0n, Si1, S2i1], [rh1, Srh1, S2rh1], mats_c1, b_c_1, U))
    h1n = u1 * h1 + (1.0 - u1) * c1

    projected = jnp.einsum("bnc,co->bno", h1n, W_proj,
                           preferred_element_type=jnp.float32) + b_proj
    output = projected.reshape(B, N)
    hidden = jnp.stack([h0n.reshape(B, N * U), h1n.reshape(B, N * U)], axis=0)
    return (output, hidden)


# speedup vs baseline: 3.8919x; 2.1867x over previous
import functools

import jax
import jax.numpy as jnp
from jax import lax
from jax.experimental import pallas as pl
from jax.experimental.pallas import tpu as pltpu
from jax.experimental.pallas import tpu_sc as plsc

N = 10000
E = 160000
U = 64
B = 16
NTILE = 16
EP = 163840
PT = EP // NTILE
C = 80
NCHUNK = PT // C
SW = 640
GROUPS = C // 16
NSLOT = 4


def _wn_body(srcs, ews, zeros_n, wn_out, accd, srcv, ewv, wnv, dval, sem):
    c = lax.axis_index("c")
    s = lax.axis_index("s")

    @pl.when(c == 0)
    def _():
        pltpu.sync_copy(srcs.at[s], srcv)
        pltpu.sync_copy(ews.at[s], ewv)

        @pl.when(s == 0)
        def _():
            pltpu.sync_copy(zeros_n, accd)

        plsc.subcore_barrier()

        def deg_chunk(k, carry):
            pltpu.sync_copy(ewv.at[k], accd.at[srcv.at[k]], add=True)
            return carry

        lax.fori_loop(0, NCHUNK, deg_chunk, 0)
        plsc.subcore_barrier()

        def wn_chunk(k, carry):
            pltpu.async_copy(accd.at[srcv.at[k]], dval, sem).wait()
            for g in range(GROUPS):
                sl = pl.ds(g * 16, 16)
                d = dval[sl]
                pos = d > 0.0
                safe = jnp.where(pos, d, 1.0)
                wnv[k, sl] = jnp.where(pos, ewv[k, sl] / safe, 0.0)
            return carry

        lax.fori_loop(0, NCHUNK, wn_chunk, 0)
        pltpu.sync_copy(wnv, wn_out.at[s])


def _spmm_body(nt, ch, x, srcs, dsts, wnb, zeros, y,
               acc, rows, srcv, dstv, wnbv, idxb, zrows, gsems, ssems):
    c = lax.axis_index("c")
    s = lax.axis_index("s")
    pltpu.sync_copy(srcs.at[s], srcv)
    pltpu.sync_copy(dsts.at[s], dstv)
    pltpu.sync_copy(zeros, zrows)

    rounds = max(nt // 2, 1)

    def do_round(b):
        off = s * SW
        pltpu.sync_copy(zrows, acc.at[pl.ds(off, 400)])

        @pl.when(s < NTILE - 1)
        def _():
            pltpu.sync_copy(zrows.at[pl.ds(0, SW - 400)],
                            acc.at[pl.ds(off + 400, SW - 400)])

        plsc.subcore_barrier()
        boff = b * N

        def fire(k, slot):
            for g in range(GROUPS):
                sl = pl.ds(g * 16, 16)
                idxb[slot, sl] = srcv[k, sl] + boff
            pltpu.async_copy(x.at[idxb.at[slot]], rows.at[slot],
                             gsems.at[slot])
            pltpu.async_copy(wnb.at[s, k], wnbv.at[slot], gsems.at[slot])

        def wait_gather(k, slot):
            pltpu.make_async_copy(x.at[idxb.at[slot]], rows.at[slot],
                                  gsems.at[slot]).wait()
            pltpu.make_async_copy(wnb.at[s, k], wnbv.at[slot],
                                  gsems.at[slot]).wait()

        def fire_scatter(k, slot):
            pltpu.async_copy(rows.at[slot], acc.at[dstv.at[k]],
                             ssems.at[slot], add=True)

        def wait_scatter(slot):
            pltpu.make_async_copy(rows.at[slot], acc.at[dstv.at[0]],
                                  ssems.at[slot]).wait()

        fire(0, 0)

        def chunk(k, carry):
            t = lax.rem(k, NSLOT)
            nslot = lax.rem(k + 1, NSLOT)

            @pl.when(k >= NSLOT - 1)
            def _():
                wait_scatter(nslot)

            @pl.when(k < NCHUNK - 1)
            def _():
                fire(k + 1, nslot)

            wait_gather(k, t)
            for j in range(C):
                wrow = wnbv[t, j]
                for cc in range(ch // 16):
                    sl = pl.ds(cc * 16, 16)
                    rows[t, j, sl] = rows[t, j, sl] * wrow
            fire_scatter(k, t)
            return carry

        lax.fori_loop(0, NCHUNK, chunk, 0)
        for slot in range(1, NSLOT):
            wait_scatter(slot)
        plsc.subcore_barrier()
        pltpu.sync_copy(acc.at[pl.ds(off, 400)], y.at[pl.ds(boff + off, 400)])

        @pl.when(s < NTILE - 1)
        def _():
            pltpu.sync_copy(acc.at[pl.ds(off + 400, SW - 400)],
                            y.at[pl.ds(boff + off + 400, SW - 400)])

    if nt == 1:
        @pl.when(c == 0)
        def _():
            do_round(0)
    else:
        for r in range(rounds):
            do_round(2 * r + c)


@functools.lru_cache(maxsize=None)
def _mesh():
    return plsc.VectorSubcoreMesh(core_axis_name="c", subcore_axis_name="s")


@functools.lru_cache(maxsize=None)
def _make_spmm(nt, ch):
    return pl.kernel(
        functools.partial(_spmm_body, nt, ch),
        out_type=jax.ShapeDtypeStruct((nt * N, ch), jnp.float32),
        mesh=_mesh(),
        scratch_types=[
            pltpu.VMEM_SHARED((N, ch), jnp.float32),
            pltpu.VMEM((NSLOT, C, ch), jnp.float32),
            pltpu.VMEM((NCHUNK, C), jnp.int32),
            pltpu.VMEM((NCHUNK, C), jnp.int32),
            pltpu.VMEM((NSLOT, C, 16), jnp.float32),
            pltpu.VMEM((NSLOT, C), jnp.int32),
            pltpu.VMEM((400, ch), jnp.float32),
            pltpu.SemaphoreType.DMA((NSLOT,)),
            pltpu.SemaphoreType.DMA((NSLOT,)),
        ],
        compiler_params=pltpu.CompilerParams(use_tc_tiling_on_sc=False),
    )


@functools.lru_cache(maxsize=None)
def _make_wn():
    return pl.kernel(
        _wn_body,
        out_type=jax.ShapeDtypeStruct((NTILE, NCHUNK, C), jnp.float32),
        mesh=_mesh(),
        scratch_types=[
            pltpu.VMEM_SHARED((N,), jnp.float32),
            pltpu.VMEM((NCHUNK, C), jnp.int32),
            pltpu.VMEM((NCHUNK, C), jnp.float32),
            pltpu.VMEM((NCHUNK, C), jnp.float32),
            pltpu.VMEM((C,), jnp.float32),
            pltpu.SemaphoreType.DMA,
        ],
        compiler_params=pltpu.CompilerParams(use_tc_tiling_on_sc=False),
    )


def _split_mats(W, ci, in_dim):
    Wr = W.reshape(in_dim, 3, -1)
    A0 = Wr[:, 0, :] - Wr[:, 2, :]
    A1 = Wr[:, 1, :]
    A2 = 2.0 * Wr[:, 2, :]
    return [(A[:ci], A[ci:]) for A in (A0, A1, A2)]


def _gconv_dense(fields_i, fields_h, mats, bias, ci):
    out = bias
    for m in range(3):
        Ai, Ah = mats[m]
        fi = fields_i[m]
        if ci == 1:
            out = out + fi[..., None] * Ai[0]
        else:
            out = out + jnp.einsum("bnc,co->bno", fi, Ai,
                                   preferred_element_type=jnp.float32)
        out = out + jnp.einsum("bnc,co->bno", fields_h[m], Ah,
                               preferred_element_type=jnp.float32)
    return out


def kernel(inputs, hidden_state, src, dst, edge_w,
           W_ru_0, b_ru_0, W_c_0, b_c_0, W_ru_1, b_ru_1, W_c_1, b_c_1,
           W_proj, b_proj):
    pad_idx = jnp.arange(EP - E, dtype=jnp.int32) % N
    src3 = jnp.concatenate([src.astype(jnp.int32), pad_idx]) \
        .reshape(NTILE, NCHUNK, C)
    dst3 = jnp.concatenate([dst.astype(jnp.int32), pad_idx]) \
        .reshape(NTILE, NCHUNK, C)
    ew3 = jnp.concatenate([edge_w, jnp.zeros((EP - E,), jnp.float32)]) \
        .reshape(NTILE, NCHUNK, C)
    zeros_n = jnp.zeros((N,), jnp.float32)
    zeros16 = jnp.zeros((400, 16), jnp.float32)
    zeros64 = jnp.zeros((400, U), jnp.float32)

    wn3 = _make_wn()(src3, ew3, zeros_n)
    wnb3 = jnp.broadcast_to(wn3[:, :, :, None], (NTILE, NCHUNK, C, 16))

    spmm64 = _make_spmm(B, U)
    spmm16 = _make_spmm(1, 16)

    def S64(xbnc):
        flat = xbnc.reshape(B * N, U)
        return spmm64(flat, src3, dst3, wnb3, zeros64).reshape(B, N, U)

    def S16(xn16):
        return spmm16(xn16, src3, dst3, wnb3, zeros16)

    x_i = inputs.reshape(B, N)
    h0 = hidden_state[0].reshape(B, N, U)
    h1 = hidden_state[1].reshape(B, N, U)

    ti = x_i.T
    Si_t = S16(ti)
    S2i_t = S16(Si_t)
    Si = Si_t.T
    S2i = S2i_t.T
    Sh0 = S64(h0)
    S2h0 = S64(Sh0)

    mats_ru0 = _split_mats(W_ru_0, 1, 1 + U)
    val = jax.nn.sigmoid(
        _gconv_dense([x_i, Si, S2i], [h0, Sh0, S2h0], mats_ru0, b_ru_0, 1))
    r0 = val[..., :U]
    u0 = val[..., U:]
    rh0 = r0 * h0
    Srh0 = S64(rh0)
    S2rh0 = S64(Srh0)
    mats_c0 = _split_mats(W_c_0, 1, 1 + U)
    c0 = jnp.tanh(
        _gconv_dense([x_i, Si, S2i], [rh0, Srh0, S2rh0], mats_c0, b_c_0, 1))
    h0n = u0 * h0 + (1.0 - u0) * c0

    Si1 = S64(h0n)
    S2i1 = S64(Si1)
    Sh1 = S64(h1)
    S2h1 = S64(Sh1)
    mats_ru1 = _split_mats(W_ru_1, U, 2 * U)
    val1 = jax.nn.sigmoid(
        _gconv_dense([h0n, Si1, S2i1], [h1, Sh1, S2h1], mats_ru1, b_ru_1, U))
    r1 = val1[..., :U]
    u1 = val1[..., U:]
    rh1 = r1 * h1
    Srh1 = S64(rh1)
    S2rh1 = S64(Srh1)
    mats_c1 = _split_mats(W_c_1, U, 2 * U)
    c1 = jnp.tanh(
        _gconv_dense([h0n, Si1, S2i1], [rh1, Srh1, S2rh1], mats_c1, b_c_1, U))
    h1n = u1 * h1 + (1.0 - u1) * c1

    projected = jnp.einsum("bnc,co->bno", h1n, W_proj,
                           preferred_element_type=jnp.float32) + b_proj
    output = projected.reshape(B, N)
    hidden = jnp.stack([h0n.reshape(B, N * U), h1n.reshape(B, N * U)], axis=0)
    return (output, hidden)


# speedup vs baseline: 4.6288x; 1.1894x over previous
import functools

import jax
import jax.numpy as jnp
from jax import lax
from jax.experimental import pallas as pl
from jax.experimental.pallas import tpu as pltpu
from jax.experimental.pallas import tpu_sc as plsc

N = 10000
E = 160000
U = 64
B = 16
NTILE = 16
EP = 163840
PT = EP // NTILE
C = 80
NCHUNK = PT // C
SW = 640
GROUPS = C // 16
NSLOT = 6
PF = 2


def _wn_body(srcs, ews, zeros_n, wn_out, accd, srcv, ewv, wnv, dval, sem):
    c = lax.axis_index("c")
    s = lax.axis_index("s")

    @pl.when(c == 0)
    def _():
        pltpu.sync_copy(srcs.at[s], srcv)
        pltpu.sync_copy(ews.at[s], ewv)

        @pl.when(s == 0)
        def _():
            pltpu.sync_copy(zeros_n, accd)

        plsc.subcore_barrier()

        def deg_chunk(k, carry):
            pltpu.sync_copy(ewv.at[k], accd.at[srcv.at[k]], add=True)
            return carry

        lax.fori_loop(0, NCHUNK, deg_chunk, 0)
        plsc.subcore_barrier()

        def wn_chunk(k, carry):
            pltpu.async_copy(accd.at[srcv.at[k]], dval, sem).wait()
            for g in range(GROUPS):
                sl = pl.ds(g * 16, 16)
                d = dval[sl]
                pos = d > 0.0
                safe = jnp.where(pos, d, 1.0)
                wnv[k, sl] = jnp.where(pos, ewv[k, sl] / safe, 0.0)
            return carry

        lax.fori_loop(0, NCHUNK, wn_chunk, 0)
        pltpu.sync_copy(wnv, wn_out.at[s])


def _spmm_body(nt, ch, x, srcs, dsts, wnb, zeros, y,
               acc, rows, srcv, dstv, wnbv, idxb, zrows, gsems, ssems):
    c = lax.axis_index("c")
    s = lax.axis_index("s")
    pltpu.sync_copy(srcs.at[s], srcv)
    pltpu.sync_copy(dsts.at[s], dstv)
    pltpu.sync_copy(zeros, zrows)

    rounds = max(nt // 2, 1)

    def do_round(b):
        off = s * SW
        pltpu.sync_copy(zrows, acc.at[pl.ds(off, 400)])

        @pl.when(s < NTILE - 1)
        def _():
            pltpu.sync_copy(zrows.at[pl.ds(0, SW - 400)],
                            acc.at[pl.ds(off + 400, SW - 400)])

        plsc.subcore_barrier()
        boff = b * N

        def fire(k, slot):
            for g in range(GROUPS):
                sl = pl.ds(g * 16, 16)
                idxb[slot, sl] = srcv[k, sl] + boff
            pltpu.async_copy(x.at[idxb.at[slot]], rows.at[slot],
                             gsems.at[slot])
            pltpu.async_copy(wnb.at[s, k], wnbv.at[slot], gsems.at[slot])

        def wait_gather(k, slot):
            pltpu.make_async_copy(x.at[idxb.at[slot]], rows.at[slot],
                                  gsems.at[slot]).wait()
            pltpu.make_async_copy(wnb.at[s, k], wnbv.at[slot],
                                  gsems.at[slot]).wait()

        def fire_scatter(k, slot):
            pltpu.async_copy(rows.at[slot], acc.at[dstv.at[k]],
                             ssems.at[slot], add=True)

        def wait_scatter(slot):
            pltpu.make_async_copy(rows.at[slot], acc.at[dstv.at[0]],
                                  ssems.at[slot]).wait()

        for kk in range(PF):
            fire(kk, kk)

        def chunk(k, carry):
            t = lax.rem(k, NSLOT)
            nslot = lax.rem(k + PF, NSLOT)

            @pl.when(k >= NSLOT - PF)
            def _():
                wait_scatter(nslot)

            @pl.when(k < NCHUNK - PF)
            def _():
                fire(k + PF, nslot)

            wait_gather(k, t)
            for j in range(C):
                wrow = wnbv[t, j]
                for cc in range(ch // 16):
                    sl = pl.ds(cc * 16, 16)
                    rows[t, j, sl] = rows[t, j, sl] * wrow
            fire_scatter(k, t)
            return carry

        lax.fori_loop(0, NCHUNK, chunk, 0)
        for kk in range(NCHUNK - (NSLOT - PF), NCHUNK):
            wait_scatter(kk % NSLOT)
        plsc.subcore_barrier()
        pltpu.sync_copy(acc.at[pl.ds(off, 400)], y.at[pl.ds(boff + off, 400)])

        @pl.when(s < NTILE - 1)
        def _():
            pltpu.sync_copy(acc.at[pl.ds(off + 400, SW - 400)],
                            y.at[pl.ds(boff + off + 400, SW - 400)])

    if nt == 1:
        @pl.when(c == 0)
        def _():
            do_round(0)
    else:
        for r in range(rounds):
            do_round(2 * r + c)


@functools.lru_cache(maxsize=None)
def _mesh():
    return plsc.VectorSubcoreMesh(core_axis_name="c", subcore_axis_name="s")


@functools.lru_cache(maxsize=None)
def _make_spmm(nt, ch):
    return pl.kernel(
        functools.partial(_spmm_body, nt, ch),
        out_type=jax.ShapeDtypeStruct((nt * N, ch), jnp.float32),
        mesh=_mesh(),
        scratch_types=[
            pltpu.VMEM_SHARED((N, ch), jnp.float32),
            pltpu.VMEM((NSLOT, C, ch), jnp.float32),
            pltpu.VMEM((NCHUNK, C), jnp.int32),
            pltpu.VMEM((NCHUNK, C), jnp.int32),
            pltpu.VMEM((NSLOT, C, 16), jnp.float32),
            pltpu.VMEM((NSLOT, C), jnp.int32),
            pltpu.VMEM((400, ch), jnp.float32),
            pltpu.SemaphoreType.DMA((NSLOT,)),
            pltpu.SemaphoreType.DMA((NSLOT,)),
        ],
        compiler_params=pltpu.CompilerParams(use_tc_tiling_on_sc=False),
    )


@functools.lru_cache(maxsize=None)
def _make_wn():
    return pl.kernel(
        _wn_body,
        out_type=jax.ShapeDtypeStruct((NTILE, NCHUNK, C), jnp.float32),
        mesh=_mesh(),
        scratch_types=[
            pltpu.VMEM_SHARED((N,), jnp.float32),
            pltpu.VMEM((NCHUNK, C), jnp.int32),
            pltpu.VMEM((NCHUNK, C), jnp.float32),
            pltpu.VMEM((NCHUNK, C), jnp.float32),
            pltpu.VMEM((C,), jnp.float32),
            pltpu.SemaphoreType.DMA,
        ],
        compiler_params=pltpu.CompilerParams(use_tc_tiling_on_sc=False),
    )


RB = 640
BN = B * N


def _gate0_body(i0, i1, i2, h0f, h1f, h2f, h, Wh, Wi, bg, u_o, rh_o):
    cat = jnp.concatenate([h0f[...], h1f[...], h2f[...]], axis=1)
    pre = jnp.dot(cat, Wh[...], preferred_element_type=jnp.float32)
    pre = pre + bg[...] + i0[...] * Wi[0:1, :] + i1[...] * Wi[1:2, :] \
        + i2[...] * Wi[2:3, :]
    val = jax.nn.sigmoid(pre)
    u_o[...] = val[:, U:]
    rh_o[...] = val[:, :U] * h[...]


def _cand0_body(i0, i1, i2, r0f, r1f, r2f, h, u, Wh, Wi, bc, hn_o):
    cat = jnp.concatenate([r0f[...], r1f[...], r2f[...]], axis=1)
    pre = jnp.dot(cat, Wh[...], preferred_element_type=jnp.float32)
    pre = pre + bc[...] + i0[...] * Wi[0:1, :] + i1[...] * Wi[1:2, :] \
        + i2[...] * Wi[2:3, :]
    cv = jnp.tanh(pre)
    uu = u[...]
    hn_o[...] = uu * h[...] + (1.0 - uu) * cv


def _gate1_body(i0f, i1f, i2f, h0f, h1f, h2f, h, W, bg, u_o, rh_o):
    cat = jnp.concatenate([i0f[...], h0f[...], i1f[...], h1f[...],
                           i2f[...], h2f[...]], axis=1)
    pre = jnp.dot(cat, W[...], preferred_element_type=jnp.float32) + bg[...]
    val = jax.nn.sigmoid(pre)
    u_o[...] = val[:, U:]
    rh_o[...] = val[:, :U] * h[...]


def _cand1_body(i0f, i1f, i2f, r0f, r1f, r2f, h, u, W, bc, wp, bp,
                hn_o, proj_o):
    cat = jnp.concatenate([i0f[...], r0f[...], i1f[...], r1f[...],
                           i2f[...], r2f[...]], axis=1)
    pre = jnp.dot(cat, W[...], preferred_element_type=jnp.float32) + bc[...]
    cv = jnp.tanh(pre)
    uu = u[...]
    hn = uu * h[...] + (1.0 - uu) * cv
    hn_o[...] = hn
    proj_o[...] = (jnp.sum(hn * wp[...], axis=1) + bp[0]).reshape(1, 1, RB)


def _fspec():
    return pl.BlockSpec((RB, U), lambda i: (i, 0))


def _sspec():
    return pl.BlockSpec((RB, 1), lambda i: (i, 0))


def _wspec(shape):
    nd = len(shape)
    return pl.BlockSpec(shape, (lambda i: (0, 0)) if nd == 2 else (lambda i: (0,)))


def _dense_call(body, in_specs, out_specs, out_shapes):
    return pl.pallas_call(
        body,
        grid=(BN // RB,),
        in_specs=in_specs,
        out_specs=out_specs,
        out_shape=out_shapes,
    )


def _split_mats(W, ci, in_dim):
    Wr = W.reshape(in_dim, 3, -1)
    A0 = Wr[:, 0, :] - Wr[:, 2, :]
    A1 = Wr[:, 1, :]
    A2 = 2.0 * Wr[:, 2, :]
    return [(A[:ci], A[ci:]) for A in (A0, A1, A2)]


def _gconv_dense(fields_i, fields_h, mats, bias, ci):
    out = bias
    for m in range(3):
        Ai, Ah = mats[m]
        fi = fields_i[m]
        if ci == 1:
            out = out + fi[..., None] * Ai[0]
        else:
            out = out + jnp.einsum("bnc,co->bno", fi, Ai,
                                   preferred_element_type=jnp.float32)
        out = out + jnp.einsum("bnc,co->bno", fields_h[m], Ah,
                               preferred_element_type=jnp.float32)
    return out


def kernel(inputs, hidden_state, src, dst, edge_w,
           W_ru_0, b_ru_0, W_c_0, b_c_0, W_ru_1, b_ru_1, W_c_1, b_c_1,
           W_proj, b_proj):
    pad_idx = jnp.arange(EP - E, dtype=jnp.int32) % N
    src3 = jnp.concatenate([src.astype(jnp.int32), pad_idx]) \
        .reshape(NTILE, NCHUNK, C)
    dst3 = jnp.concatenate([dst.astype(jnp.int32), pad_idx]) \
        .reshape(NTILE, NCHUNK, C)
    ew3 = jnp.concatenate([edge_w, jnp.zeros((EP - E,), jnp.float32)]) \
        .reshape(NTILE, NCHUNK, C)
    zeros_n = jnp.zeros((N,), jnp.float32)
    zeros16 = jnp.zeros((400, 16), jnp.float32)
    zeros64 = jnp.zeros((400, U), jnp.float32)

    wn3 = _make_wn()(src3, ew3, zeros_n)
    wnb3 = jnp.broadcast_to(wn3[:, :, :, None], (NTILE, NCHUNK, C, 16))

    spmm64 = _make_spmm(B, U)
    spmm16 = _make_spmm(1, 16)

    def S64(flat):
        return spmm64(flat, src3, dst3, wnb3, zeros64)

    def S16(xn16):
        return spmm16(xn16, src3, dst3, wnb3, zeros16)

    f32 = jnp.float32
    x_i = inputs.reshape(B, N)
    h0 = hidden_state[0].reshape(BN, U)
    h1 = hidden_state[1].reshape(BN, U)

    ti = x_i.T
    Si_t = S16(ti)
    S2i_t = S16(Si_t)
    i0c = x_i.reshape(BN, 1)
    i1c = Si_t.T.reshape(BN, 1)
    i2c = S2i_t.T.reshape(BN, 1)
    Sh0 = S64(h0)
    S2h0 = S64(Sh0)

    m_ru0 = _split_mats(W_ru_0, 1, 1 + U)
    Whg0 = jnp.concatenate([m[1] for m in m_ru0], axis=0)
    Wig0 = jnp.concatenate([m[0] for m in m_ru0], axis=0)
    fspecs3 = [_sspec()] * 3 + [_fspec()] * 3
    u0, rh0 = _dense_call(
        _gate0_body,
        fspecs3 + [_fspec(), _wspec((3 * U, 2 * U)), _wspec((3, 2 * U)),
                   _wspec((1, 2 * U))],
        [_fspec(), _fspec()],
        [jax.ShapeDtypeStruct((BN, U), f32)] * 2,
    )(i0c, i1c, i2c, h0, Sh0, S2h0, h0, Whg0, Wig0, b_ru_0.reshape(1, 2 * U))

    Srh0 = S64(rh0)
    S2rh0 = S64(Srh0)
    m_c0 = _split_mats(W_c_0, 1, 1 + U)
    Whc0 = jnp.concatenate([m[1] for m in m_c0], axis=0)
    Wic0 = jnp.concatenate([m[0] for m in m_c0], axis=0)
    h0n = _dense_call(
        _cand0_body,
        fspecs3 + [_fspec(), _fspec(), _wspec((3 * U, U)), _wspec((3, U)),
                   _wspec((1, U))],
        _fspec(),
        jax.ShapeDtypeStruct((BN, U), f32),
    )(i0c, i1c, i2c, rh0, Srh0, S2rh0, h0, u0, Whc0, Wic0,
      b_c_0.reshape(1, U))

    Si1 = S64(h0n)
    S2i1 = S64(Si1)
    Sh1 = S64(h1)
    S2h1 = S64(Sh1)
    m_ru1 = _split_mats(W_ru_1, U, 2 * U)
    Wg1 = jnp.concatenate([m[j] for m in m_ru1 for j in (0, 1)], axis=0)
    u1, rh1 = _dense_call(
        _gate1_body,
        [_fspec()] * 7 + [_wspec((6 * U, 2 * U)), _wspec((1, 2 * U))],
        [_fspec(), _fspec()],
        [jax.ShapeDtypeStruct((BN, U), f32)] * 2,
    )(h0n, Si1, S2i1, h1, Sh1, S2h1, h1, Wg1, b_ru_1.reshape(1, 2 * U))

    Srh1 = S64(rh1)
    S2rh1 = S64(Srh1)
    m_c1 = _split_mats(W_c_1, U, 2 * U)
    Wc1 = jnp.concatenate([m[j] for m in m_c1 for j in (0, 1)], axis=0)
    h1n, proj = _dense_call(
        _cand1_body,
        [_fspec()] * 8 + [_wspec((6 * U, U)), _wspec((1, U)),
                          _wspec((1, U)), _wspec((1,))],
        [_fspec(), pl.BlockSpec((1, 1, RB), lambda i: (i, 0, 0))],
        [jax.ShapeDtypeStruct((BN, U), f32),
         jax.ShapeDtypeStruct((BN // RB, 1, RB), f32)],
    )(h0n, Si1, S2i1, rh1, Srh1, S2rh1, h1, u1, Wc1, b_c_1.reshape(1, U),
      W_proj.reshape(1, U), b_proj)

    output = proj.reshape(B, N)
    hidden = jnp.stack([h0n.reshape(B, N * U), h1n.reshape(B, N * U)], axis=0)
    return (output, hidden)
